# baseline (speedup 1.0000x reference)
import functools

import jax
import jax.numpy as jnp
from jax import lax
from jax.experimental import pallas as pl
from jax.experimental.pallas import tpu as pltpu
from jax.experimental.pallas import tpu_sc as plsc

_ROWS = 8192
_D = 4096
_N_TEC = 32
_W = _D // _N_TEC
_CG = _W // 16
_CHUNK = 256
_NCHUNK = _ROWS // _CHUNK
_K = _ROWS // 2

_PASSES = ((23, None, 8), (15, 23, 8), (7, 15, 8), (0, 7, 7))


def _sc_body(x_ref, o_ref, stage, ostage, hist, vthr, sem, osem):
    core = lax.axis_index("c")
    sub = lax.axis_index("s")
    tec = core * 16 + sub
    col0 = tec * _W
    iota = lax.iota(jnp.int32, 16)
    ones = jnp.ones((16,), jnp.int32)

    kk = [jnp.full((16,), _K, jnp.int32) for _ in range(_CG)]
    prefix = [jnp.zeros((16,), jnp.int32) for _ in range(_CG)]

    for shift, rshift, nbits in _PASSES:
        nbins = 1 << nbits
        def _zero(i, _):
            hist[i] = jnp.zeros((16,), jnp.int32)
            return 0
        lax.fori_loop(0, _CG * nbins, _zero, 0)

        def _chunk(ci, carry):
            cp = pltpu.make_async_copy(
                x_ref.at[pl.ds(ci * _CHUNK, _CHUNK), pl.ds(col0, _W)],
                stage, sem)
            cp.start()
            cp.wait()

            def _row(r, c2):
                for cg in range(_CG):
                    v = stage[r, pl.ds(cg * 16, 16)]
                    u = plsc.bitcast(jnp.abs(v), jnp.int32)
                    b = lax.shift_right_logical(u, shift) & (nbins - 1)
                    dst = b + (cg * nbins)
                    if rshift is None:
                        plsc.addupdate_scatter(hist, [dst, iota], ones)
                    else:
                        ok = lax.shift_right_logical(u, rshift) == prefix[cg]
                        plsc.addupdate_scatter(hist, [dst, iota], ones,
                                               mask=ok)
                return c2
            lax.fori_loop(0, _CHUNK, _row, 0)
            return carry
        lax.fori_loop(0, _NCHUNK, _chunk, 0)

        for cg in range(_CG):
            def _scan(bi, carry):
                run, bucket, rbase = carry
                h = hist[cg * nbins + bi]
                new = run + h
                hit = (run < kk[cg]) & (new >= kk[cg])
                bucket = jnp.where(hit, bi, bucket)
                rbase = jnp.where(hit, run, rbase)
                return new, bucket, rbase
            z = jnp.zeros((16,), jnp.int32)
            _, bucket, rbase = lax.fori_loop(0, nbins, _scan, (z, z, z))
            kk[cg] = kk[cg] - rbase
            prefix[cg] = lax.shift_left(prefix[cg], nbits) | bucket

    for cg in range(_CG):
        vthr[cg] = prefix[cg]

    def _mchunk(ci, carry):
        cp = pltpu.make_async_copy(
            x_ref.at[pl.ds(ci * _CHUNK, _CHUNK), pl.ds(col0, _W)],
            stage, sem)
        cp.start()
        cp.wait()

        def _row(r, c2):
            for cg in range(_CG):
                v = stage[r, pl.ds(cg * 16, 16)]
                u = plsc.bitcast(jnp.abs(v), jnp.int32)
                ostage[r, pl.ds(cg * 16, 16)] = jnp.where(
                    u <= vthr[cg], jnp.zeros((16,), jnp.float32), v)
            return c2
        lax.fori_loop(0, _CHUNK, _row, 0)

        ocp = pltpu.make_async_copy(
            ostage,
            o_ref.at[pl.ds(ci * _CHUNK, _CHUNK), pl.ds(col0, _W)], osem)
        ocp.start()
        ocp.wait()
        return carry
    lax.fori_loop(0, _NCHUNK, _mchunk, 0)


def kernel(inputs, interpret=False):
    shape = inputs.shape
    x2 = inputs.reshape(_ROWS, _D)
    mesh = plsc.VectorSubcoreMesh(core_axis_name="c", subcore_axis_name="s")
    fn = pl.kernel(
        _sc_body,
        compiler_params=pltpu.CompilerParams(needs_layout_passes=False,
                                             use_tc_tiling_on_sc=False),
        out_type=jax.ShapeDtypeStruct((_ROWS, _D), jnp.float32),
        mesh=mesh,
        scratch_types=[
            pltpu.VMEM((_CHUNK, _W), jnp.float32),
            pltpu.VMEM((_CHUNK, _W), jnp.float32),
            pltpu.VMEM((_CG * 256, 16), jnp.int32),
            pltpu.VMEM((_CG, 16), jnp.int32),
            pltpu.SemaphoreType.DMA,
            pltpu.SemaphoreType.DMA,
        ],
        interpret=interpret,
    )
    return fn(x2).reshape(shape)


if False:
    import reference as R
    x = jax.random.normal(jax.random.key(0), (2, 4096, 4096), jnp.float32)
    got = kernel_sc(x, interpret=True)
    ref = R.reference(x)
    print("exact", bool(jnp.array_equal(got, ref)),
          "resvar", float(jnp.var(got - ref)))


# speedup vs baseline: 2.9042x; 2.9042x over previous
import functools

import jax
import jax.numpy as jnp
from jax import lax
from jax.experimental import pallas as pl
from jax.experimental.pallas import tpu as pltpu
from jax.experimental.pallas import tpu_sc as plsc

_ROWS = 8192
_D = 4096
_N_TEC = 32
_W = _D // _N_TEC
_CG = _W // 16
_CHUNK = 256
_NCHUNK = _ROWS // _CHUNK
_K = _ROWS // 2

_PASSES = ((23, None, 8), (15, 23, 8), (7, 15, 8), (0, 7, 7))


def _sc_body(x_ref, o_ref, stage, ostage, hist, vthr, sem, osem):
    core = lax.axis_index("c")
    sub = lax.axis_index("s")
    tec = core * 16 + sub
    col0 = tec * _W
    iota = lax.iota(jnp.int32, 16)
    ones = jnp.ones((16,), jnp.int32)

    kk = [jnp.full((16,), _K, jnp.int32) for _ in range(_CG)]
    prefix = [jnp.zeros((16,), jnp.int32) for _ in range(_CG)]

    for shift, rshift, nbits in _PASSES:
        nbins = 1 << nbits
        def _zero(i, _):
            hist[i] = jnp.zeros((16,), jnp.int32)
            return 0
        lax.fori_loop(0, _CG * nbins, _zero, 0)

        def _chunk(ci, carry):
            cp = pltpu.make_async_copy(
                x_ref.at[pl.ds(ci * _CHUNK, _CHUNK), pl.ds(col0, _W)],
                stage, sem)
            cp.start()
            cp.wait()

            @plsc.parallel_loop(0, _CHUNK, unroll=4)
            def _row(r):
                for cg in range(_CG):
                    v = stage[r, pl.ds(cg * 16, 16)]
                    u = plsc.bitcast(jnp.abs(v), jnp.int32)
                    b = lax.shift_right_logical(u, shift) & (nbins - 1)
                    dst = b + (cg * nbins)
                    if rshift is None:
                        plsc.addupdate_scatter(hist, [dst, iota], ones)
                    else:
                        ok = lax.shift_right_logical(u, rshift) == prefix[cg]
                        plsc.addupdate_scatter(hist, [dst, iota], ones,
                                               mask=ok)
            return carry
        lax.fori_loop(0, _NCHUNK, _chunk, 0)

        for cg in range(_CG):
            def _scan(bi, carry):
                run, bucket, rbase = carry
                h = hist[cg * nbins + bi]
                new = run + h
                hit = (run < kk[cg]) & (new >= kk[cg])
                bucket = jnp.where(hit, bi, bucket)
                rbase = jnp.where(hit, run, rbase)
                return new, bucket, rbase
            z = jnp.zeros((16,), jnp.int32)
            _, bucket, rbase = lax.fori_loop(0, nbins, _scan, (z, z, z))
            kk[cg] = kk[cg] - rbase
            prefix[cg] = lax.shift_left(prefix[cg], nbits) | bucket

    for cg in range(_CG):
        vthr[cg] = prefix[cg]

    def _mchunk(ci, carry):
        cp = pltpu.make_async_copy(
            x_ref.at[pl.ds(ci * _CHUNK, _CHUNK), pl.ds(col0, _W)],
            stage, sem)
        cp.start()
        cp.wait()

        @plsc.parallel_loop(0, _CHUNK, unroll=4)
        def _row(r):
            for cg in range(_CG):
                v = stage[r, pl.ds(cg * 16, 16)]
                u = plsc.bitcast(jnp.abs(v), jnp.int32)
                ostage[r, pl.ds(cg * 16, 16)] = jnp.where(
                    u <= vthr[cg], jnp.zeros((16,), jnp.float32), v)

        ocp = pltpu.make_async_copy(
            ostage,
            o_ref.at[pl.ds(ci * _CHUNK, _CHUNK), pl.ds(col0, _W)], osem)
        ocp.start()
        ocp.wait()
        return carry
    lax.fori_loop(0, _NCHUNK, _mchunk, 0)


def kernel(inputs, interpret=False):
    shape = inputs.shape
    x2 = inputs.reshape(_ROWS, _D)
    mesh = plsc.VectorSubcoreMesh(core_axis_name="c", subcore_axis_name="s")
    fn = pl.kernel(
        _sc_body,
        compiler_params=pltpu.CompilerParams(needs_layout_passes=False,
                                             use_tc_tiling_on_sc=False),
        out_type=jax.ShapeDtypeStruct((_ROWS, _D), jnp.float32),
        mesh=mesh,
        scratch_types=[
            pltpu.VMEM((_CHUNK, _W), jnp.float32),
            pltpu.VMEM((_CHUNK, _W), jnp.float32),
            pltpu.VMEM((_CG * 256, 16), jnp.int32),
            pltpu.VMEM((_CG, 16), jnp.int32),
            pltpu.SemaphoreType.DMA,
            pltpu.SemaphoreType.DMA,
        ],
        interpret=interpret,
    )
    return fn(x2).reshape(shape)


if False:
    import reference as R
    x = jax.random.normal(jax.random.key(0), (2, 4096, 4096), jnp.float32)
    got = kernel_sc(x, interpret=True)
    ref = R.reference(x)
    print("exact", bool(jnp.array_equal(got, ref)),
          "resvar", float(jnp.var(got - ref)))


# speedup vs baseline: 8.6574x; 2.9810x over previous
import functools

import jax
import jax.numpy as jnp
from jax import lax
from jax.experimental import pallas as pl
from jax.experimental.pallas import tpu as pltpu
from jax.experimental.pallas import tpu_sc as plsc

_TARGET_RATE = 0.5
_ROWS = 8192
_D = 4096
_K = int(_TARGET_RATE * float(_ROWS))

_BLOCK_COLS = 256
_TC_COLS = 3072
_SC_COLS = _D - _TC_COLS

_N_TEC = 32
_W = _SC_COLS // _N_TEC
_CG = _W // 16
_CHUNK = 256
_NCHUNK = _ROWS // _CHUNK

_PASSES = ((23, None, 8), (15, 23, 8), (7, 15, 8), (0, 7, 7))


def _count_i16(mask):
    acc = mask.astype(jnp.int16)
    r = acc.shape[0]
    while r > 16:
        r //= 2
        acc = acc[:r] + acc[r:]
    s = jnp.sum(acc.astype(jnp.int32), axis=0, keepdims=True)
    return s.astype(jnp.int16)


def _tc_select_mask_kernel(k, x_ref, o_ref):
    x = x_ref[...]
    u = lax.bitcast_convert_type(jnp.abs(x), jnp.int32)
    i16 = jnp.int16

    h = ((u >> 15) - 32768).astype(i16)
    prefix = jnp.full((1, x.shape[1]), -32768, dtype=i16)
    for b in range(15, -1, -1):
        bit = i16(-32768) if b == 15 else i16(1 << b)
        cand = prefix | i16((1 << b) - 1)
        cnt = _count_i16(h <= cand)
        prefix = jnp.where(cnt >= i16(k), prefix, prefix ^ bit)
    hp = prefix

    base = _count_i16(h < hp)
    m = h == hp
    k2 = i16(k) - base

    lo = ((u & 0x7FFF) - 32768).astype(i16)
    lm = jnp.where(m, lo, i16(0))
    prefix2 = jnp.full((1, x.shape[1]), -32768, dtype=i16)
    for b in range(14, -1, -1):
        cand = prefix2 | i16((1 << b) - 1)
        cnt = _count_i16(lm <= cand)
        prefix2 = jnp.where(cnt >= k2, prefix2, prefix2 | i16(1 << b))

    v = ((hp.astype(jnp.int32) + 32768) << 15) | (
        prefix2.astype(jnp.int32) + 32768)
    o_ref[...] = jnp.where(u <= v, jnp.zeros_like(x), x)


def _tc_main(x2):
    nb = _TC_COLS // _BLOCK_COLS
    return pl.pallas_call(
        functools.partial(_tc_select_mask_kernel, _K),
        grid=(nb,),
        in_specs=[pl.BlockSpec((_ROWS, _BLOCK_COLS), lambda j: (0, j))],
        out_specs=pl.BlockSpec((_ROWS, _BLOCK_COLS), lambda j: (0, j)),
        out_shape=jax.ShapeDtypeStruct((_ROWS, _D), x2.dtype),
    )(x2)


def _tc_apply_kernel(big_ref, x_ref, t_ref, o_ref):
    x = x_ref[...]
    u = lax.bitcast_convert_type(jnp.abs(x), jnp.int32)
    thr = t_ref[0]
    o_ref[...] = jnp.where(u <= thr, jnp.zeros_like(x), x)


def _tc_apply(big, x2, thr3):
    nb = _SC_COLS // _BLOCK_COLS
    off = _TC_COLS // _BLOCK_COLS
    return pl.pallas_call(
        _tc_apply_kernel,
        grid=(nb,),
        in_specs=[
            pl.BlockSpec(memory_space=pl.ANY),
            pl.BlockSpec((_ROWS, _BLOCK_COLS), lambda j: (0, j + off)),
            pl.BlockSpec((1, 1, _BLOCK_COLS), lambda j: (j, 0, 0)),
        ],
        out_specs=pl.BlockSpec((_ROWS, _BLOCK_COLS), lambda j: (0, j + off)),
        out_shape=jax.ShapeDtypeStruct((_ROWS, _D), x2.dtype),
        input_output_aliases={0: 0},
    )(big, x2, thr3)


def _sc_thresh_body(x_ref, o_ref, stage, hist, thr_s, sem, osem):
    core = lax.axis_index("c")
    sub = lax.axis_index("s")
    tec = core * 16 + sub
    col0 = tec * _W
    iota = lax.iota(jnp.int32, 16)
    ones = jnp.ones((16,), jnp.int32)

    kk = [jnp.full((16,), _K, jnp.int32) for _ in range(_CG)]
    prefix = [jnp.zeros((16,), jnp.int32) for _ in range(_CG)]

    for shift, rshift, nbits in _PASSES:
        nbins = 1 << nbits

        def _zero(i, _):
            hist[i] = jnp.zeros((16,), jnp.int32)
            return 0
        lax.fori_loop(0, _CG * nbins, _zero, 0)

        def _chunk(ci, carry):
            cp = pltpu.make_async_copy(
                x_ref.at[pl.ds(ci * _CHUNK, _CHUNK), pl.ds(col0, _W)],
                stage, sem)
            cp.start()
            cp.wait()

            @plsc.parallel_loop(0, _CHUNK, unroll=4)
            def _row(r):
                for cg in range(_CG):
                    v = stage[r, pl.ds(cg * 16, 16)]
                    u = plsc.bitcast(jnp.abs(v), jnp.int32)
                    b = lax.shift_right_logical(u, shift) & (nbins - 1)
                    dst = b + (cg * nbins)
                    if rshift is None:
                        plsc.addupdate_scatter(hist, [dst, iota], ones)
                    else:
                        ok = lax.shift_right_logical(u, rshift) == prefix[cg]
                        plsc.addupdate_scatter(hist, [dst, iota], ones,
                                               mask=ok)
            return carry
        lax.fori_loop(0, _NCHUNK, _chunk, 0)

        for cg in range(_CG):
            def _scan(bi, carry):
                run, bucket, rbase = carry
                h = hist[cg * nbins + bi]
                new = run + h
                hit = (run < kk[cg]) & (new >= kk[cg])
                bucket = jnp.where(hit, bi, bucket)
                rbase = jnp.where(hit, run, rbase)
                return new, bucket, rbase
            z = jnp.zeros((16,), jnp.int32)
            _, bucket, rbase = lax.fori_loop(0, nbins, _scan, (z, z, z))
            kk[cg] = kk[cg] - rbase
            prefix[cg] = lax.shift_left(prefix[cg], nbits) | bucket

    for cg in range(_CG):
        thr_s[pl.ds(cg * 16, 16)] = prefix[cg]
    ocp = pltpu.make_async_copy(thr_s, o_ref.at[pl.ds(col0, _W)], osem)
    ocp.start()
    ocp.wait()


def _sc_thresholds(x_sc):
    mesh = plsc.VectorSubcoreMesh(core_axis_name="c", subcore_axis_name="s")
    fn = pl.kernel(
        _sc_thresh_body,
        out_type=jax.ShapeDtypeStruct((_SC_COLS,), jnp.int32),
        mesh=mesh,
        compiler_params=pltpu.CompilerParams(needs_layout_passes=False,
                                             use_tc_tiling_on_sc=False),
        scratch_types=[
            pltpu.VMEM((_CHUNK, _W), jnp.float32),
            pltpu.VMEM((_CG * 256, 16), jnp.int32),
            pltpu.VMEM((_W,), jnp.int32),
            pltpu.SemaphoreType.DMA,
            pltpu.SemaphoreType.DMA,
        ],
    )
    return fn(x_sc)


def kernel(inputs):
    shape = inputs.shape
    x2 = inputs.reshape(_ROWS, _D)
    x_sc = lax.slice(x2, (0, _TC_COLS), (_ROWS, _D))
    thr = _sc_thresholds(x_sc)
    thr3 = thr.reshape(_SC_COLS // _BLOCK_COLS, 1, _BLOCK_COLS)
    big = _tc_main(x2)
    out = _tc_apply(big, x2, thr3)
    return out.reshape(shape)
